# Initial kernel scaffold; baseline (speedup 1.0000x reference)
import jax
import jax.numpy as jnp
from jax.experimental import pallas as pl


def kernel(x, gaze, fc1_w, fc1_b, fc1_gamma, fc1_beta, gw, gb, g_gamma, g_beta, fc2_w, fc2_b, fc2_gamma, fc2_beta):
    raise NotImplementedError("write your pallas kernel here")


# speedup vs baseline: 9.3851x; 9.3851x over previous
import functools

import jax
import jax.numpy as jnp
from jax import lax
from jax.experimental import pallas as pl

B, C, H, W = 4, 192, 32, 32
N = H * W
K = 9
C2 = 2 * C
G = 4
EPS_BN = 1e-5
_PREC = lax.Precision.DEFAULT


def _bn_cols(hs, gamma_col, beta_col):
    cnt = float(len(hs) * hs[0].shape[1])
    s = hs[0]
    for h in hs[1:]:
        s = s + h
    mean = jnp.sum(s, axis=1, keepdims=True) / cnt
    v = None
    for h in hs:
        d = h - mean
        vv = jnp.sum(d * d, axis=1, keepdims=True)
        v = vv if v is None else v + vv
    var = v / cnt
    inv = gamma_col / jnp.sqrt(var + EPS_BN)
    return [(h - mean) * inv + beta_col for h in hs]


def _stage_a_kernel(x_ref, w_ref, b_ref, gam_ref, bet_ref, xr_ref, xn_ref):
    w = w_ref[...]
    hs = [
        lax.dot_general(w, x_ref[b], (((1,), (0,)), ((), ())),
                        preferred_element_type=jnp.float32, precision=_PREC)
        + b_ref[...]
        for b in range(B)
    ]
    hs = _bn_cols(hs, gam_ref[...], bet_ref[...])
    for b in range(B):
        xr = hs[b]
        xr_ref[b] = xr
        nsq = jnp.sum(xr * xr, axis=0, keepdims=True)
        nrm = jnp.maximum(jnp.sqrt(nsq), 1e-12)
        xn_ref[b] = xr / nrm


def _stage_b_kernel(xn_ref, xnt_ref, xr_ref, gzr_ref, gzc_ref, out_ref):
    xnb = xn_ref[0]
    xnbt = xnt_ref[0]
    xrb = xr_ref[0]
    gzr = gzr_ref[0]
    gzc = gzc_ref[0]

    inner = lax.dot_general(xnbt, xnbt, (((1,), (1,)), ((), ())),
                            preferred_element_type=jnp.float32, precision=_PREC)
    xsq_row = jnp.sum(xnb * xnb, axis=0, keepdims=True)
    xsq_col = jnp.sum(xnbt * xnbt, axis=1, keepdims=True)
    dist = xsq_col + (-2.0) * inner + xsq_row

    g_inner = (-2.0) * (gzc * gzr)
    gdist = gzc * gzc + g_inner + gzr * gzr
    gmax = jnp.max(gzr)
    gmin = jnp.min(gzr)
    gnorm = (gzc - gmin) / (gmax - gmin)
    dist = dist + gdist * gnorm

    hi = xrb.astype(jnp.bfloat16)
    lo = (xrb - hi.astype(jnp.float32)).astype(jnp.bfloat16)
    iota_j = lax.broadcasted_iota(jnp.int32, (N, N), 1)
    runmax = jnp.full((C, N), -jnp.inf, jnp.float32)
    for _ in range(K):
        rowmin = jnp.min(dist, axis=1, keepdims=True)
        am = jnp.min(jnp.where(dist == rowmin, iota_j, N), axis=1,
                     keepdims=True)
        oh = (iota_j == am)
        ohb = oh.astype(jnp.bfloat16)
        gat = (
            lax.dot_general(hi, ohb, (((1,), (1,)), ((), ())),
                            preferred_element_type=jnp.float32)
            + lax.dot_general(lo, ohb, (((1,), (1,)), ((), ())),
                              preferred_element_type=jnp.float32)
        )
        runmax = jnp.maximum(runmax, gat)
        dist = jnp.where(oh, jnp.inf, dist)
    out_ref[0] = runmax - xrb


def _gelu_exact(x):
    return 0.5 * x * (1.0 + lax.erf(x * 0.7071067811865476))


def _stage_c_kernel(xr_ref, mr_ref, x_ref, we_ref, wo_ref, gb_ref, gg_ref,
                    gbe_ref, w2_ref, b2_ref, g2_ref, be2_ref, out_ref):
    Cg = C // G
    hs = []
    for b in range(B):
        parts = []
        for g in range(G):
            xr_s = xr_ref[b, g * Cg:(g + 1) * Cg, :]
            mr_s = mr_ref[b, g * Cg:(g + 1) * Cg, :]
            p = (
                lax.dot_general(we_ref[g], xr_s, (((1,), (0,)), ((), ())),
                                preferred_element_type=jnp.float32,
                                precision=_PREC)
                + lax.dot_general(wo_ref[g], mr_s, (((1,), (0,)), ((), ())),
                                  preferred_element_type=jnp.float32,
                                  precision=_PREC)
            )
            parts.append(p)
        hs.append(jnp.concatenate(parts, axis=0) + gb_ref[...])
    hs = _bn_cols(hs, gg_ref[...], gbe_ref[...])
    hs = [_gelu_exact(h) for h in hs]
    os = [
        lax.dot_general(w2_ref[...], h, (((1,), (0,)), ((), ())),
                        preferred_element_type=jnp.float32, precision=_PREC)
        + b2_ref[...]
        for h in hs
    ]
    os = _bn_cols(os, g2_ref[...], be2_ref[...])
    for b in range(B):
        out_ref[b] = os[b] + x_ref[b]


def kernel(x, gaze, fc1_w, fc1_b, fc1_gamma, fc1_beta, gw, gb, g_gamma,
           g_beta, fc2_w, fc2_b, fc2_gamma, fc2_beta):
    xf = x.reshape(B, C, N)
    gzr = gaze.reshape(B, 1, N)
    gzc = jnp.swapaxes(gzr, 1, 2)

    xr, xn = pl.pallas_call(
        _stage_a_kernel,
        out_shape=(
            jax.ShapeDtypeStruct((B, C, N), jnp.float32),
            jax.ShapeDtypeStruct((B, C, N), jnp.float32),
        ),
    )(xf, fc1_w, fc1_b.reshape(C, 1), fc1_gamma.reshape(C, 1),
      fc1_beta.reshape(C, 1))

    xnt = jnp.swapaxes(xn, 1, 2)

    maxrel = pl.pallas_call(
        _stage_b_kernel,
        grid=(B,),
        in_specs=[
            pl.BlockSpec((1, C, N), lambda b: (b, 0, 0)),
            pl.BlockSpec((1, N, C), lambda b: (b, 0, 0)),
            pl.BlockSpec((1, C, N), lambda b: (b, 0, 0)),
            pl.BlockSpec((1, 1, N), lambda b: (b, 0, 0)),
            pl.BlockSpec((1, N, 1), lambda b: (b, 0, 0)),
        ],
        out_specs=pl.BlockSpec((1, C, N), lambda b: (b, 0, 0)),
        out_shape=jax.ShapeDtypeStruct((B, C, N), jnp.float32),
    )(xn, xnt, xr, gzr, gzc)

    wg = gw.reshape(G, C2 // G, C2 // G)
    we = wg[:, :, 0::2]
    wo = wg[:, :, 1::2]

    out = pl.pallas_call(
        _stage_c_kernel,
        out_shape=jax.ShapeDtypeStruct((B, C, N), jnp.float32),
    )(xr, maxrel, xf, we, wo, gb.reshape(C2, 1), g_gamma.reshape(C2, 1),
      g_beta.reshape(C2, 1), fc2_w, fc2_b.reshape(C, 1),
      fc2_gamma.reshape(C, 1), fc2_beta.reshape(C, 1))

    return (out.reshape(B, C, H, W), gaze)
